# baseline (speedup 1.0000x reference)
import functools
import jax
import jax.numpy as jnp
from jax import lax
from jax.experimental import pallas as pl
from jax.experimental.pallas import tpu as pltpu
from jax.experimental.pallas import tpu_sc as plsc

_N = 1000
_ROWS = 16384
_L = 16
_NW = 32
_RPW = _ROWS // _NW
_NB = _RPW // _L
_BLK = _L * _N
_NEG = -1e7


def _insert5(t, v):
    t1, t2, t3, t4, t5 = t
    m = jnp.maximum(t1, v); v = jnp.minimum(t1, v); t1 = m
    m = jnp.maximum(t2, v); v = jnp.minimum(t2, v); t2 = m
    m = jnp.maximum(t3, v); v = jnp.minimum(t3, v); t3 = m
    m = jnp.maximum(t4, v); v = jnp.minimum(t4, v); t4 = m
    t5 = jnp.maximum(t5, v)
    return (t1, t2, t3, t4, t5)


def _merge5(a, b):
    a1, a2, a3, a4, a5 = a
    b1, b2, b3, b4, b5 = b
    mn = jnp.minimum
    mx = jnp.maximum
    c1 = mx(a1, b1)
    c2 = mx(mx(mn(a1, b1), a2), b2)
    c3 = mx(mx(mn(a1, b2), mn(a2, b1)), mx(a3, b3))
    c4 = mx(mx(mx(mn(a1, b3), mn(a2, b2)), mx(mn(a3, b1), a4)), b4)
    c5 = mx(mx(mx(mn(a1, b4), mn(a2, b3)), mx(mn(a3, b2), mn(a4, b1))),
            mx(a5, b5))
    return (c1, c2, c3, c4, c5)


def _make_sc_kernel():
    mesh = plsc.VectorSubcoreMesh(core_axis_name="c", subcore_axis_name="s")

    @functools.partial(
        pl.kernel,
        mesh=mesh,
        out_type=jax.ShapeDtypeStruct((_ROWS,), jnp.float32),
        scratch_types=[
            pltpu.VMEM((_L, _N), jnp.float32),
            pltpu.VMEM((_L, _N), jnp.float32),
            pltpu.VMEM((_RPW,), jnp.int32),
            pltpu.VMEM((_RPW,), jnp.float32),
            pltpu.SemaphoreType.DMA,
            pltpu.SemaphoreType.DMA,
        ],
        compiler_params=pltpu.CompilerParams(use_tc_tiling_on_sc=True,
                                             needs_layout_passes=False),
    )
    def _sc_kernel(logits_hbm, labels_hbm, out_hbm, buf0, buf1, labs_v,
                   out_v, sem0, sem1):
        wid = lax.axis_index("s") * 2 + lax.axis_index("c")
        base = wid * _RPW
        pltpu.sync_copy(labels_hbm.at[pl.ds(base, _RPW)], labs_v)

        lane = lax.broadcasted_iota(jnp.int32, (_L,), 0)
        ninf = jnp.full((_L,), -jnp.inf, jnp.float32)
        zero16 = jnp.zeros((_L,), jnp.int32)

        def start_copy(b, buf, sem):
            src = logits_hbm.at[pl.ds(base + b * _L, _L)]
            pltpu.async_copy(src, buf, sem)

        def wait_copy(b, buf, sem):
            src = logits_hbm.at[pl.ds(base + b * _L, _L)]
            pltpu.make_async_copy(src, buf, sem).wait()

        def compute(b, buf):
            labs = labs_v[pl.ds(b * _L, _L)]
            true_v = plsc.load_gather(buf, [lane, labs])
            plsc.store_scatter(buf, [lane, labs],
                               jnp.full((_L,), _NEG, jnp.float32))

            t0 = ((ninf,) * 5, (ninf,) * 5, zero16)

            @plsc.parallel_loop(0, _N, step=2, unroll=4, carry=t0)
            def cls_loop(c, t):
                ta, tb, idx = t
                va = plsc.load_gather(buf, [lane, idx])
                vb = plsc.load_gather(buf, [lane, idx + 1])
                ta = _insert5(ta, va)
                tb = _insert5(tb, vb)
                return (ta, tb, idx + 2)

            ta, tb, _ = cls_loop
            t1, t2, t3, t4, t5 = _merge5(ta, tb)
            base_m = true_v + 1.0
            loss = jnp.maximum(base_m - t1, 0.0)
            loss = loss + jnp.maximum(base_m - t2, 0.0)
            loss = loss + jnp.maximum(base_m - t3, 0.0)
            loss = loss + jnp.maximum(base_m - t4, 0.0)
            loss = loss + jnp.maximum(base_m - t5, 0.0)
            out_v[pl.ds(b * _L, _L)] = loss

        start_copy(0, buf0, sem0)

        def outer(i, carry):
            b = 2 * i
            start_copy(b + 1, buf1, sem1)
            wait_copy(b, buf0, sem0)
            compute(b, buf0)

            @pl.when(b + 2 < _NB)
            def _():
                start_copy(b + 2, buf0, sem0)

            wait_copy(b + 1, buf1, sem1)
            compute(b + 1, buf1)
            return carry

        lax.fori_loop(0, _NB // 2, outer, 0)
        pltpu.sync_copy(out_v, out_hbm.at[pl.ds(base, _RPW)])

    return _sc_kernel


_SC_KERNEL = _make_sc_kernel()


def kernel(logits, labels):
    return _SC_KERNEL(logits, labels.astype(jnp.int32))


# speedup vs baseline: 1.9981x; 1.9981x over previous
import functools
import jax
import jax.numpy as jnp
from jax import lax
from jax.experimental import pallas as pl
from jax.experimental.pallas import tpu as pltpu
from jax.experimental.pallas import tpu_sc as plsc

_N = 1000
_ROWS = 16384
_L = 16
_NW = 32
_RPW = _ROWS // _NW
_NB = _RPW // _L
_NC = _N // _L
_CAND = 5 * _L
_STR = 85
_NEG = -1e7


def _insert5(t, v):
    t1, t2, t3, t4, t5 = t
    m = jnp.maximum(t1, v); v = jnp.minimum(t1, v); t1 = m
    m = jnp.maximum(t2, v); v = jnp.minimum(t2, v); t2 = m
    m = jnp.maximum(t3, v); v = jnp.minimum(t3, v); t3 = m
    m = jnp.maximum(t4, v); v = jnp.minimum(t4, v); t4 = m
    t5 = jnp.maximum(t5, v)
    return (t1, t2, t3, t4, t5)


def _make_sc_kernel():
    mesh = plsc.VectorSubcoreMesh(core_axis_name="c", subcore_axis_name="s")

    @functools.partial(
        pl.kernel,
        mesh=mesh,
        out_type=jax.ShapeDtypeStruct((_ROWS,), jnp.float32),
        scratch_types=[
            pltpu.VMEM((_L, _N), jnp.float32),
            pltpu.VMEM((_L, _N), jnp.float32),
            pltpu.VMEM((_L * _STR,), jnp.float32),
            pltpu.VMEM((_RPW,), jnp.int32),
            pltpu.VMEM((_RPW,), jnp.float32),
            pltpu.SemaphoreType.DMA,
            pltpu.SemaphoreType.DMA,
        ],
        compiler_params=pltpu.CompilerParams(use_tc_tiling_on_sc=True,
                                             needs_layout_passes=False),
    )
    def _sc_kernel(logits_hbm, labels_hbm, out_hbm, buf0, buf1, cand,
                   labs_v, out_v, sem0, sem1):
        wid = lax.axis_index("s") * 2 + lax.axis_index("c")
        base = wid * _RPW
        pltpu.sync_copy(labels_hbm.at[pl.ds(base, _RPW)], labs_v)

        lane = lax.broadcasted_iota(jnp.int32, (_L,), 0)
        ninf = jnp.full((_L,), -jnp.inf, jnp.float32)
        tail_idx = jnp.minimum(jnp.full((_L,), _NC * _L, jnp.int32) + lane,
                               _N - 1)
        tail_msk = lane < (_N - _NC * _L)
        cand_base = lane * _STR

        def start_copy(b, buf, sem):
            src = logits_hbm.at[pl.ds(base + b * _L, _L)]
            pltpu.async_copy(src, buf, sem)

        def wait_copy(b, buf, sem):
            src = logits_hbm.at[pl.ds(base + b * _L, _L)]
            pltpu.make_async_copy(src, buf, sem).wait()

        def compute(b, buf):
            labs = labs_v[pl.ds(b * _L, _L)]
            true_v = plsc.load_gather(buf, [lane, labs])
            plsc.store_scatter(buf, [lane, labs],
                               jnp.full((_L,), _NEG, jnp.float32))

            def row_body(r, carry):
                @plsc.parallel_loop(0, _NC, step=1, unroll=2,
                                    carry=(ninf,) * 5)
                def chunks(j, t):
                    return _insert5(t, buf[r, pl.ds(j * _L, _L)])

                vt = plsc.load_gather(buf, [jnp.full((_L,), r, jnp.int32),
                                            tail_idx])
                vt = jnp.where(tail_msk, vt, -jnp.inf)
                t1, t2, t3, t4, t5 = _insert5(chunks, vt)
                rb = r * _STR
                plsc.store_scatter(cand, [rb + lane], t1)
                plsc.store_scatter(cand, [rb + _L + lane], t2)
                plsc.store_scatter(cand, [rb + 2 * _L + lane], t3)
                plsc.store_scatter(cand, [rb + 3 * _L + lane], t4)
                plsc.store_scatter(cand, [rb + 4 * _L + lane], t5)
                return carry

            lax.fori_loop(0, _L, row_body, 0)

            @plsc.parallel_loop(0, _CAND, step=1, unroll=4,
                                carry=((ninf,) * 5, cand_base))
            def ph2(c, t):
                tt, idx = t
                v = plsc.load_gather(cand, [idx])
                return (_insert5(tt, v), idx + 1)

            (t1, t2, t3, t4, t5), _ = ph2
            base_m = true_v + 1.0
            loss = jnp.maximum(base_m - t1, 0.0)
            loss = loss + jnp.maximum(base_m - t2, 0.0)
            loss = loss + jnp.maximum(base_m - t3, 0.0)
            loss = loss + jnp.maximum(base_m - t4, 0.0)
            loss = loss + jnp.maximum(base_m - t5, 0.0)
            out_v[pl.ds(b * _L, _L)] = loss

        start_copy(0, buf0, sem0)

        def outer(i, carry):
            b = 2 * i
            start_copy(b + 1, buf1, sem1)
            wait_copy(b, buf0, sem0)
            compute(b, buf0)

            @pl.when(b + 2 < _NB)
            def _():
                start_copy(b + 2, buf0, sem0)

            wait_copy(b + 1, buf1, sem1)
            compute(b + 1, buf1)
            return carry

        lax.fori_loop(0, _NB // 2, outer, 0)
        pltpu.sync_copy(out_v, out_hbm.at[pl.ds(base, _RPW)])

    return _sc_kernel


_SC_KERNEL = _make_sc_kernel()


def kernel(logits, labels):
    return _SC_KERNEL(logits, labels.astype(jnp.int32))


# speedup vs baseline: 1.9982x; 1.0000x over previous
import functools
import jax
import jax.numpy as jnp
from jax import lax
from jax.experimental import pallas as pl
from jax.experimental.pallas import tpu as pltpu
from jax.experimental.pallas import tpu_sc as plsc

_N = 1000
_ROWS = 16384
_L = 16
_NW = 32
_RPW = _ROWS // _NW
_NB = _RPW // _L
_NC = _N // _L
_CAND = 5 * _L
_STR = 85
_NEG = -1e7


def _insert5(t, v):
    t1, t2, t3, t4, t5 = t
    m = jnp.maximum(t1, v); v = jnp.minimum(t1, v); t1 = m
    m = jnp.maximum(t2, v); v = jnp.minimum(t2, v); t2 = m
    m = jnp.maximum(t3, v); v = jnp.minimum(t3, v); t3 = m
    m = jnp.maximum(t4, v); v = jnp.minimum(t4, v); t4 = m
    t5 = jnp.maximum(t5, v)
    return (t1, t2, t3, t4, t5)


def _make_sc_kernel():
    mesh = plsc.VectorSubcoreMesh(core_axis_name="c", subcore_axis_name="s")

    @functools.partial(
        pl.kernel,
        mesh=mesh,
        out_type=jax.ShapeDtypeStruct((_ROWS,), jnp.float32),
        scratch_types=[
            pltpu.VMEM((_L, _N), jnp.float32),
            pltpu.VMEM((_L, _N), jnp.float32),
            pltpu.VMEM((_L * _STR,), jnp.float32),
            pltpu.VMEM((_RPW,), jnp.int32),
            pltpu.VMEM((_RPW,), jnp.float32),
            pltpu.SemaphoreType.DMA,
            pltpu.SemaphoreType.DMA,
        ],
        compiler_params=pltpu.CompilerParams(use_tc_tiling_on_sc=True,
                                             needs_layout_passes=False,
                                             disable_bounds_checks=True,
                                             disable_semaphore_checks=True,
                                             skip_device_barrier=True),
    )
    def _sc_kernel(logits_hbm, labels_hbm, out_hbm, buf0, buf1, cand,
                   labs_v, out_v, sem0, sem1):
        wid = lax.axis_index("s") * 2 + lax.axis_index("c")
        base = wid * _RPW
        pltpu.sync_copy(labels_hbm.at[pl.ds(base, _RPW)], labs_v)

        lane = lax.broadcasted_iota(jnp.int32, (_L,), 0)
        ninf = jnp.full((_L,), -jnp.inf, jnp.float32)
        tail_idx = jnp.minimum(jnp.full((_L,), _NC * _L, jnp.int32) + lane,
                               _N - 1)
        tail_msk = lane < (_N - _NC * _L)
        cand_base = lane * _STR

        def start_copy(b, buf, sem):
            src = logits_hbm.at[pl.ds(base + b * _L, _L)]
            pltpu.async_copy(src, buf, sem)

        def wait_copy(b, buf, sem):
            src = logits_hbm.at[pl.ds(base + b * _L, _L)]
            pltpu.make_async_copy(src, buf, sem).wait()

        def compute(b, buf):
            labs = labs_v[pl.ds(b * _L, _L)]
            true_v = plsc.load_gather(buf, [lane, labs])
            plsc.store_scatter(buf, [lane, labs],
                               jnp.full((_L,), _NEG, jnp.float32))

            def row_body(r, carry):
                @plsc.parallel_loop(0, _NC, step=1, unroll=2,
                                    carry=(ninf,) * 5)
                def chunks(j, t):
                    return _insert5(t, buf[r, pl.ds(j * _L, _L)])

                vt = plsc.load_gather(buf, [jnp.full((_L,), r, jnp.int32),
                                            tail_idx])
                vt = jnp.where(tail_msk, vt, -jnp.inf)
                t1, t2, t3, t4, t5 = _insert5(chunks, vt)
                rb = r * _STR
                plsc.store_scatter(cand, [rb + lane], t1)
                plsc.store_scatter(cand, [rb + _L + lane], t2)
                plsc.store_scatter(cand, [rb + 2 * _L + lane], t3)
                plsc.store_scatter(cand, [rb + 3 * _L + lane], t4)
                plsc.store_scatter(cand, [rb + 4 * _L + lane], t5)
                return carry

            lax.fori_loop(0, _L, row_body, 0)

            @plsc.parallel_loop(0, _CAND, step=1, unroll=4,
                                carry=((ninf,) * 5, cand_base))
            def ph2(c, t):
                tt, idx = t
                v = plsc.load_gather(cand, [idx])
                return (_insert5(tt, v), idx + 1)

            (t1, t2, t3, t4, t5), _ = ph2
            base_m = true_v + 1.0
            loss = jnp.maximum(base_m - t1, 0.0)
            loss = loss + jnp.maximum(base_m - t2, 0.0)
            loss = loss + jnp.maximum(base_m - t3, 0.0)
            loss = loss + jnp.maximum(base_m - t4, 0.0)
            loss = loss + jnp.maximum(base_m - t5, 0.0)
            out_v[pl.ds(b * _L, _L)] = loss

        start_copy(0, buf0, sem0)

        def outer(i, carry):
            b = 2 * i
            start_copy(b + 1, buf1, sem1)
            wait_copy(b, buf0, sem0)
            compute(b, buf0)

            @pl.when(b + 2 < _NB)
            def _():
                start_copy(b + 2, buf0, sem0)

            wait_copy(b + 1, buf1, sem1)
            compute(b + 1, buf1)
            return carry

        lax.fori_loop(0, _NB // 2, outer, 0)
        pltpu.sync_copy(out_v, out_hbm.at[pl.ds(base, _RPW)])

    return _sc_kernel


_SC_KERNEL = _make_sc_kernel()


def kernel(logits, labels):
    return _SC_KERNEL(logits, labels.astype(jnp.int32))


# speedup vs baseline: 2.2743x; 1.1382x over previous
import functools
import jax
import jax.numpy as jnp
from jax import lax
from jax.experimental import pallas as pl
from jax.experimental.pallas import tpu as pltpu
from jax.experimental.pallas import tpu_sc as plsc

_N = 1000
_ROWS = 16384
_L = 16
_NW = 32
_RPW = _ROWS // _NW
_NB = _RPW // _L
_NC = _N // _L
_CAND = 5 * _L
_STR = 85
_NEG = -1e7


def _insert5(t, v):
    t1, t2, t3, t4, t5 = t
    m = jnp.maximum(t1, v); v = jnp.minimum(t1, v); t1 = m
    m = jnp.maximum(t2, v); v = jnp.minimum(t2, v); t2 = m
    m = jnp.maximum(t3, v); v = jnp.minimum(t3, v); t3 = m
    m = jnp.maximum(t4, v); v = jnp.minimum(t4, v); t4 = m
    t5 = jnp.maximum(t5, v)
    return (t1, t2, t3, t4, t5)


def _make_sc_kernel():
    mesh = plsc.VectorSubcoreMesh(core_axis_name="c", subcore_axis_name="s")

    @functools.partial(
        pl.kernel,
        mesh=mesh,
        out_type=jax.ShapeDtypeStruct((_ROWS,), jnp.float32),
        scratch_types=[
            pltpu.VMEM((_L, _N), jnp.float32),
            pltpu.VMEM((_L, _N), jnp.float32),
            pltpu.VMEM((_L * _STR,), jnp.float32),
            pltpu.VMEM((_RPW,), jnp.int32),
            pltpu.VMEM((_RPW,), jnp.float32),
            pltpu.SemaphoreType.DMA,
            pltpu.SemaphoreType.DMA,
        ],
        compiler_params=pltpu.CompilerParams(use_tc_tiling_on_sc=True,
                                             needs_layout_passes=False,
                                             disable_bounds_checks=True,
                                             disable_semaphore_checks=True,
                                             skip_device_barrier=True),
    )
    def _sc_kernel(logits_hbm, labels_hbm, out_hbm, buf0, buf1, cand,
                   labs_v, out_v, sem0, sem1):
        wid = lax.axis_index("s") * 2 + lax.axis_index("c")
        base = wid * _RPW
        pltpu.sync_copy(labels_hbm.at[pl.ds(base, _RPW)], labs_v)

        lane = lax.broadcasted_iota(jnp.int32, (_L,), 0)
        ninf = jnp.full((_L,), -jnp.inf, jnp.float32)
        tail_idx = jnp.minimum(jnp.full((_L,), _NC * _L, jnp.int32) + lane,
                               _N - 1)
        tail_msk = lane < (_N - _NC * _L)
        cand_base = lane * _STR

        def start_copy(b, buf, sem):
            src = logits_hbm.at[pl.ds(base + b * _L, _L)]
            pltpu.async_copy(src, buf, sem)

        def wait_copy(b, buf, sem):
            src = logits_hbm.at[pl.ds(base + b * _L, _L)]
            pltpu.make_async_copy(src, buf, sem).wait()

        def compute(b, buf):
            labs = labs_v[pl.ds(b * _L, _L)]
            true_v = plsc.load_gather(buf, [lane, labs])
            plsc.store_scatter(buf, [lane, labs],
                               jnp.full((_L,), _NEG, jnp.float32))

            def row_body(rr, carry):
                r0 = 2 * rr
                r1 = r0 + 1

                @plsc.parallel_loop(0, _NC, step=1, unroll=2,
                                    carry=((ninf,) * 5, (ninf,) * 5))
                def chunks(j, t):
                    ta, tb = t
                    ta = _insert5(ta, buf[r0, pl.ds(j * _L, _L)])
                    tb = _insert5(tb, buf[r1, pl.ds(j * _L, _L)])
                    return (ta, tb)

                ta, tb = chunks
                for r, t in ((r0, ta), (r1, tb)):
                    vt = plsc.load_gather(
                        buf, [jnp.full((_L,), r, jnp.int32), tail_idx])
                    vt = jnp.where(tail_msk, vt, -jnp.inf)
                    t1, t2, t3, t4, t5 = _insert5(t, vt)
                    rb = r * _STR
                    plsc.store_scatter(cand, [rb + lane], t1)
                    plsc.store_scatter(cand, [rb + _L + lane], t2)
                    plsc.store_scatter(cand, [rb + 2 * _L + lane], t3)
                    plsc.store_scatter(cand, [rb + 3 * _L + lane], t4)
                    plsc.store_scatter(cand, [rb + 4 * _L + lane], t5)
                return carry

            lax.fori_loop(0, _L // 2, row_body, 0)

            @plsc.parallel_loop(0, _CAND, step=1, unroll=4,
                                carry=((ninf,) * 5, cand_base))
            def ph2(c, t):
                tt, idx = t
                v = plsc.load_gather(cand, [idx])
                return (_insert5(tt, v), idx + 1)

            (t1, t2, t3, t4, t5), _ = ph2
            base_m = true_v + 1.0
            loss = jnp.maximum(base_m - t1, 0.0)
            loss = loss + jnp.maximum(base_m - t2, 0.0)
            loss = loss + jnp.maximum(base_m - t3, 0.0)
            loss = loss + jnp.maximum(base_m - t4, 0.0)
            loss = loss + jnp.maximum(base_m - t5, 0.0)
            out_v[pl.ds(b * _L, _L)] = loss

        start_copy(0, buf0, sem0)

        def outer(i, carry):
            b = 2 * i
            start_copy(b + 1, buf1, sem1)
            wait_copy(b, buf0, sem0)
            compute(b, buf0)

            @pl.when(b + 2 < _NB)
            def _():
                start_copy(b + 2, buf0, sem0)

            wait_copy(b + 1, buf1, sem1)
            compute(b + 1, buf1)
            return carry

        lax.fori_loop(0, _NB // 2, outer, 0)
        pltpu.sync_copy(out_v, out_hbm.at[pl.ds(base, _RPW)])

    return _sc_kernel


_SC_KERNEL = _make_sc_kernel()


def kernel(logits, labels):
    return _SC_KERNEL(logits, labels.astype(jnp.int32))
